# baseline (speedup 1.0000x reference)
import functools

import jax
import jax.numpy as jnp
from jax import lax
from jax.experimental import pallas as pl
from jax.experimental.pallas import tpu as pltpu
from jax.experimental.pallas import tpu_sc as plsc

VOCAB = 1_000_000
EMBED_DIM = 64
BATCH = 4096
HIST = 200

NUM_WORKERS = 32
ROWS_PER_W = BATCH // NUM_WORKERS
IDX_PER_W = ROWS_PER_W * HIST
CHUNK = 128
NCHUNKS = IDX_PER_W // CHUNK
FIRE = 8

VBLK = 8000


def _tc_scores_body(w_ref, b_ref, emb_ref, out_ref):
    s = lax.dot_general(
        w_ref[...], emb_ref[...],
        dimension_numbers=(((1,), (1,)), ((), ())),
        preferred_element_type=jnp.float32,
    )
    out_ref[...] = ((s + b_ref[0, 0]) * (1.0 / HIST)).reshape(1, 1, VBLK)


def _tc_scores(emb_table, w_row, b11):
    return pl.pallas_call(
        _tc_scores_body,
        grid=(VOCAB // VBLK,),
        in_specs=[
            pl.BlockSpec((1, EMBED_DIM), lambda i: (0, 0)),
            pl.BlockSpec((1, 1), lambda i: (0, 0)),
            pl.BlockSpec((VBLK, EMBED_DIM), lambda i: (i, 0)),
        ],
        out_specs=pl.BlockSpec((1, 1, VBLK), lambda i: (i, 0, 0)),
        out_shape=jax.ShapeDtypeStruct((VOCAB // VBLK, 1, VBLK), jnp.float32),
    )(w_row, b11, emb_table)


def _sc_pool_body(x_hbm, scores_hbm, out_hbm, xv, gv, ov, sem):
    cid = lax.axis_index("c")
    sid = lax.axis_index("s")
    wid = sid * 2 + cid
    base = pl.multiple_of(wid * IDX_PER_W, IDX_PER_W)

    pltpu.sync_copy(x_hbm.at[pl.ds(base, IDX_PER_W)], xv)

    @pl.loop(0, NCHUNKS // FIRE)
    def _(i):
        copies = []
        for u in range(FIRE):
            j = i * FIRE + u
            off = pl.multiple_of(j * CHUNK, CHUNK)
            copies.append(
                pltpu.async_copy(
                    scores_hbm.at[xv.at[pl.ds(off, CHUNK)]],
                    gv.at[pl.ds(off, CHUNK)],
                    sem,
                )
            )
        for cp in copies:
            cp.wait()

    lanes = lax.iota(jnp.int32, 16)

    def hsum_all_lanes(v):
        for d in (8, 4, 2, 1):
            v = v + lax.gather(
                v,
                (lanes ^ d)[:, None],
                dimension_numbers=lax.GatherDimensionNumbers(
                    offset_dims=(),
                    collapsed_slice_dims=(0,),
                    start_index_map=(0,),
                ),
                slice_sizes=(1,),
                mode=lax.GatherScatterMode.PROMISE_IN_BOUNDS,
            )
        return v

    @pl.loop(0, ROWS_PER_W // 16)
    def _(g):
        gb = pl.multiple_of(g * 16 * HIST, 8)
        rowsum = jnp.zeros((16,), jnp.float32)
        for rr in range(16):
            rb = gb + rr * HIST
            acc = gv[pl.ds(rb, 16)]
            for k in range(1, 12):
                acc = acc + gv[pl.ds(rb + k * 16, 16)]
            tail = gv[pl.ds(rb + HIST - 16, 16)]
            acc = acc + jnp.where(lanes >= 8, tail, 0.0)
            rowsum = jnp.where(lanes == rr, hsum_all_lanes(acc), rowsum)
        ov[pl.ds(pl.multiple_of(g * 16, 16), 16)] = rowsum

    pltpu.sync_copy(ov, out_hbm.at[pl.ds(wid * ROWS_PER_W, ROWS_PER_W)])


@functools.partial(
    pl.kernel,
    out_type=jax.ShapeDtypeStruct((BATCH,), jnp.float32),
    mesh=plsc.VectorSubcoreMesh(core_axis_name="c", subcore_axis_name="s",
                                num_cores=2, num_subcores=16),
    scratch_types=[
        pltpu.VMEM((IDX_PER_W,), jnp.int32),
        pltpu.VMEM((IDX_PER_W,), jnp.float32),
        pltpu.VMEM((ROWS_PER_W,), jnp.float32),
        pltpu.SemaphoreType.DMA,
    ],
)
def _sc_pool(x_hbm, scores_hbm, out_hbm, xv, gv, ov, sem):
    _sc_pool_body(x_hbm, scores_hbm, out_hbm, xv, gv, ov, sem)


def kernel(x, emb_table, W, b):
    b11 = b.reshape(1, 1)
    scores = _tc_scores(emb_table, W, b11).reshape(VOCAB)
    xflat = x.astype(jnp.int32).reshape(BATCH * HIST)
    out = _sc_pool(xflat, scores)
    return out.reshape(BATCH, 1)


# speedup vs baseline: 1.0175x; 1.0175x over previous
import functools

import jax
import jax.numpy as jnp
from jax import lax
from jax.experimental import pallas as pl
from jax.experimental.pallas import tpu as pltpu
from jax.experimental.pallas import tpu_sc as plsc

VOCAB = 1_000_000
EMBED_DIM = 64
BATCH = 4096
HIST = 200

NUM_WORKERS = 32
ROWS_PER_W = BATCH // NUM_WORKERS
IDX_PER_W = ROWS_PER_W * HIST
CHUNK = 128
NCHUNKS = IDX_PER_W // CHUNK
FIRE = 8

VBLK = 8000


def _tc_scores_body(w_ref, b_ref, emb_ref, out_ref):
    s = lax.dot_general(
        w_ref[...], emb_ref[...],
        dimension_numbers=(((1,), (1,)), ((), ())),
        preferred_element_type=jnp.float32,
    )
    out_ref[...] = ((s + b_ref[0, 0]) * (1.0 / HIST)).reshape(1, 1, VBLK)


def _tc_scores(emb_table, w_row, b11):
    return pl.pallas_call(
        _tc_scores_body,
        grid=(VOCAB // VBLK,),
        in_specs=[
            pl.BlockSpec((1, EMBED_DIM), lambda i: (0, 0)),
            pl.BlockSpec((1, 1), lambda i: (0, 0)),
            pl.BlockSpec((VBLK, EMBED_DIM), lambda i: (i, 0)),
        ],
        out_specs=pl.BlockSpec((1, 1, VBLK), lambda i: (i, 0, 0)),
        out_shape=jax.ShapeDtypeStruct((VOCAB // VBLK, 1, VBLK), jnp.float32),
    )(w_row, b11, emb_table)


def _sc_pool_body(x_hbm, scores_hbm, out_hbm, xv, gv, ov, sem):
    cid = lax.axis_index("c")
    sid = lax.axis_index("s")
    wid = sid * 2 + cid
    rbase = pl.multiple_of(wid * ROWS_PER_W, ROWS_PER_W)

    pltpu.sync_copy(x_hbm.at[pl.ds(rbase, ROWS_PER_W), :], xv)

    @pl.loop(0, ROWS_PER_W // FIRE)
    def _(i):
        copies = []
        for u in range(FIRE):
            r = i * FIRE + u
            rb = pl.multiple_of(r * HIST, 8)
            copies.append(
                pltpu.async_copy(
                    scores_hbm.at[xv.at[r, pl.ds(0, CHUNK)]],
                    gv.at[pl.ds(rb, CHUNK)],
                    sem,
                )
            )
            copies.append(
                pltpu.async_copy(
                    scores_hbm.at[xv.at[r, pl.ds(CHUNK, HIST - CHUNK)]],
                    gv.at[pl.ds(rb + CHUNK, HIST - CHUNK)],
                    sem,
                )
            )
        for cp in copies:
            cp.wait()

    lanes = lax.iota(jnp.int32, 16)

    def hsum_all_lanes(v):
        for d in (8, 4, 2, 1):
            v = v + lax.gather(
                v,
                (lanes ^ d)[:, None],
                dimension_numbers=lax.GatherDimensionNumbers(
                    offset_dims=(),
                    collapsed_slice_dims=(0,),
                    start_index_map=(0,),
                ),
                slice_sizes=(1,),
                mode=lax.GatherScatterMode.PROMISE_IN_BOUNDS,
            )
        return v

    @pl.loop(0, ROWS_PER_W // 16)
    def _(g):
        gb = pl.multiple_of(g * 16 * HIST, 8)
        rowsum = jnp.zeros((16,), jnp.float32)
        for rr in range(16):
            rb = gb + rr * HIST
            acc = gv[pl.ds(rb, 16)]
            for k in range(1, 12):
                acc = acc + gv[pl.ds(rb + k * 16, 16)]
            tail = gv[pl.ds(rb + HIST - 16, 16)]
            acc = acc + jnp.where(lanes >= 8, tail, 0.0)
            rowsum = jnp.where(lanes == rr, hsum_all_lanes(acc), rowsum)
        ov[pl.ds(pl.multiple_of(g * 16, 16), 16)] = rowsum

    pltpu.sync_copy(ov, out_hbm.at[pl.ds(wid * ROWS_PER_W, ROWS_PER_W)])


@functools.partial(
    pl.kernel,
    out_type=jax.ShapeDtypeStruct((BATCH,), jnp.float32),
    mesh=plsc.VectorSubcoreMesh(core_axis_name="c", subcore_axis_name="s",
                                num_cores=2, num_subcores=16),
    scratch_types=[
        pltpu.VMEM((ROWS_PER_W, HIST), jnp.int32),
        pltpu.VMEM((IDX_PER_W,), jnp.float32),
        pltpu.VMEM((ROWS_PER_W,), jnp.float32),
        pltpu.SemaphoreType.DMA,
    ],
)
def _sc_pool(x_hbm, scores_hbm, out_hbm, xv, gv, ov, sem):
    _sc_pool_body(x_hbm, scores_hbm, out_hbm, xv, gv, ov, sem)


def kernel(x, emb_table, W, b):
    b11 = b.reshape(1, 1)
    scores = _tc_scores(emb_table, W, b11).reshape(VOCAB)
    out = _sc_pool(x.astype(jnp.int32), scores)
    return out.reshape(BATCH, 1)
